# speedup vs baseline: 1.4050x; 1.4050x over previous
import jax
import jax.numpy as jnp
from jax import lax
from jax.experimental import pallas as pl

_D = 1000


def _body(idx_ref, out_ref):
    iota = lax.broadcasted_iota(jnp.int32, out_ref.shape, 1)
    out_ref[...] = (idx_ref[...] == iota).astype(jnp.float32)


def kernel(tensor):
    B, F = tensor.shape
    N = B * F
    flat = tensor.astype(jnp.int32).reshape(N, 1)
    R = 1024
    out = pl.pallas_call(
        _body,
        grid=(N // R,),
        in_specs=[pl.BlockSpec((R, 1), lambda i: (i, 0))],
        out_specs=pl.BlockSpec((R, _D), lambda i: (i, 0)),
        out_shape=jax.ShapeDtypeStruct((N, _D), jnp.float32),
    )(flat)
    return out.reshape(B, F * _D)


# speedup vs baseline: 2.4108x; 1.7159x over previous
import jax
import jax.numpy as jnp
from jax import lax
from jax.experimental import pallas as pl

_D = 1000
_F = 26
_BT = 128


def _body(idx_ref, out_ref):
    iota = lax.broadcasted_iota(jnp.int32, (_BT, _D), 1)
    for i in range(_F):
        col = idx_ref[:, i : i + 1]
        out_ref[:, i * _D : (i + 1) * _D] = (col == iota).astype(jnp.float32)


def kernel(tensor):
    B, F = tensor.shape
    idx = tensor.astype(jnp.int32)
    out = pl.pallas_call(
        _body,
        grid=(B // _BT,),
        in_specs=[pl.BlockSpec((_BT, F), lambda i: (i, 0))],
        out_specs=pl.BlockSpec((_BT, F * _D), lambda i: (i, 0)),
        out_shape=jax.ShapeDtypeStruct((B, F * _D), jnp.float32),
    )(idx)
    return out
